# Initial kernel scaffold; baseline (speedup 1.0000x reference)
import jax
import jax.numpy as jnp
from jax.experimental import pallas as pl


def kernel(feat, edge_index, W1_0, b1_0, W2_0, b2_0, Ws1_0, Ws2_0, W1_1, b1_1, W2_1, b2_1, Ws1_1, Ws2_1):
    raise NotImplementedError("write your pallas kernel here")


# speedup vs baseline: 1.9589x; 1.9589x over previous
import functools

import jax
import jax.numpy as jnp
from jax import lax
from jax.experimental import pallas as pl
from jax.experimental.pallas import tpu as pltpu
from jax.experimental.pallas import tpu_sc as plsc

N = 10000
R = 3
E = 320000
D = 128
A = 16

NC = 2
NS = 16
NW = NC * NS
CHUNK = 128
NCH = -(-E // (NW * CHUNK))
NCH = NCH + (NCH % 2)
E_PAD = NW * NCH * CHUNK
ROWS_PER_TILE = 632
N_ACC = ROWS_PER_TILE * NS


def _sc_segment_sums(x0, x1, x2, srcs, dsts, zeros):
    mesh = plsc.VectorSubcoreMesh(core_axis_name="c", subcore_axis_name="s")

    @functools.partial(
        pl.kernel,
        mesh=mesh,
        out_type=jax.ShapeDtypeStruct((NC, R, N_ACC, D), jnp.float32),
        scratch_types=[
            pltpu.VMEM((NCH, CHUNK), jnp.int32),
            pltpu.VMEM((NCH, CHUNK), jnp.int32),
            pltpu.VMEM((CHUNK, D), jnp.float32),
            pltpu.VMEM_SHARED((N_ACC, D), jnp.float32),
            pltpu.SemaphoreType.DMA,
        ],
    )
    def seg(x0_hbm, x1_hbm, x2_hbm, srcs_hbm, dsts_hbm, zeros_hbm, out_hbm,
            src_v, dst_v, rows_v, acc, sem):
        c = lax.axis_index("c")
        s = lax.axis_index("s")
        wid = s * NC + c
        row0 = s * ROWS_PER_TILE
        tables = (x0_hbm, x1_hbm, x2_hbm)
        for r in range(R):
            pltpu.sync_copy(zeros_hbm.at[pl.ds(row0, ROWS_PER_TILE)],
                            acc.at[pl.ds(row0, ROWS_PER_TILE)])
            pltpu.sync_copy(srcs_hbm.at[r, wid], src_v)
            pltpu.sync_copy(dsts_hbm.at[r, wid], dst_v)
            plsc.subcore_barrier()

            def chunk_body(j, carry):
                pltpu.async_copy(tables[r].at[src_v.at[j]], rows_v, sem).wait()
                pltpu.sync_copy(rows_v, acc.at[dst_v.at[j]], add=True)
                return carry

            lax.fori_loop(0, NCH, chunk_body, 0)
            plsc.subcore_barrier()
            pltpu.sync_copy(acc.at[pl.ds(row0, ROWS_PER_TILE)],
                            out_hbm.at[c, r, pl.ds(row0, ROWS_PER_TILE)])

    return seg(x0, x1, x2, srcs, dsts, zeros)


def _bdot(a, b):
    return jnp.dot(a.astype(jnp.bfloat16), b.astype(jnp.bfloat16),
                   preferred_element_type=jnp.float32)

def _tc_layer_body(x_ref, a0_ref, a1_ref, w1_ref, b1_ref, w2_ref, b2_ref,
                   ws1_ref, ws2_ref, out_ref, *, last):
    hs = []
    logits = []
    for r in range(R):
        xr = x_ref[0] if x_ref.shape[0] == 1 else x_ref[r]
        t = xr + a0_ref[r] + a1_ref[r]
        h = jnp.maximum(_bdot(t, w1_ref[:]) + b1_ref[:], 0.0)
        h = jnp.maximum(_bdot(h, w2_ref[:]) + b2_ref[:], 0.0)
        s = jnp.tanh(_bdot(h, ws1_ref[r]))
        logits.append(_bdot(s, ws2_ref[r]))
        hs.append(h)
    m = jnp.maximum(jnp.maximum(logits[0], logits[1]), logits[2])
    e = [jnp.exp(l - m) for l in logits]
    den = e[0] + e[1] + e[2]
    for rp in range(R):
        a = e[rp] / den
        o = a[:, 0:1] * hs[0] + a[:, 1:2] * hs[1] + a[:, 2:3] * hs[2]
        if last:
            out_ref[:, rp, :] = o
        else:
            out_ref[rp] = o


def _tc_layer(x, agg0, agg1, W1, b1, W2, b2, Ws1, Ws2, *, last):
    B = 512
    grid = (-(-N // B),)
    rx = x.shape[0]
    in_specs = [
        pl.BlockSpec((rx, B, D), lambda i: (0, i, 0)),
        pl.BlockSpec((R, B, D), lambda i: (0, i, 0)),
        pl.BlockSpec((R, B, D), lambda i: (0, i, 0)),
        pl.BlockSpec((D, D), lambda i: (0, 0)),
        pl.BlockSpec((1, D), lambda i: (0, 0)),
        pl.BlockSpec((D, D), lambda i: (0, 0)),
        pl.BlockSpec((1, D), lambda i: (0, 0)),
        pl.BlockSpec((R, D, A), lambda i: (0, 0, 0)),
        pl.BlockSpec((R, A, R), lambda i: (0, 0, 0)),
    ]
    if last:
        out_spec = pl.BlockSpec((B, R, D), lambda i: (i, 0, 0))
        out_shape = jax.ShapeDtypeStruct((N, R, D), jnp.float32)
    else:
        out_spec = pl.BlockSpec((R, B, D), lambda i: (0, i, 0))
        out_shape = jax.ShapeDtypeStruct((R, N, D), jnp.float32)
    return pl.pallas_call(
        functools.partial(_tc_layer_body, last=last),
        grid=grid,
        in_specs=in_specs,
        out_specs=out_spec,
        out_shape=out_shape,
        compiler_params=pltpu.CompilerParams(
            dimension_semantics=("arbitrary",)),
    )(x, agg0, agg1, W1, b1.reshape(1, D), W2, b2.reshape(1, D), Ws1, Ws2)


def kernel(feat, edge_index, W1_0, b1_0, W2_0, b2_0, Ws1_0, Ws2_0,
           W1_1, b1_1, W2_1, b2_1, Ws1_1, Ws2_1):
    src = edge_index[:, 0, :]
    dst = edge_index[:, 1, :]
    pad = E_PAD - E
    src = jnp.pad(src, ((0, 0), (0, pad)), constant_values=0)
    dst = jnp.pad(dst, ((0, 0), (0, pad)), constant_values=N)
    srcs = src.reshape(R, NW, NCH, CHUNK)
    dsts = dst.reshape(R, NW, NCH, CHUNK)
    zeros = jnp.zeros((N_ACC, D), jnp.float32)

    agg = _sc_segment_sums(feat, feat, feat, srcs, dsts, zeros)
    agg = agg[:, :, :N, :]
    h = _tc_layer(feat[None], agg[0], agg[1], W1_0, b1_0, W2_0, b2_0,
                  Ws1_0, Ws2_0, last=False)
    agg = _sc_segment_sums(h[0], h[1], h[2], srcs, dsts, zeros)
    agg = agg[:, :, :N, :]
    out = _tc_layer(h, agg[0], agg[1], W1_1, b1_1, W2_1, b2_1,
                    Ws1_1, Ws2_1, last=True)
    return out


# speedup vs baseline: 2.1717x; 1.1086x over previous
import functools

import jax
import jax.numpy as jnp
from jax import lax
from jax.experimental import pallas as pl
from jax.experimental.pallas import tpu as pltpu
from jax.experimental.pallas import tpu_sc as plsc

N = 10000
R = 3
E = 320000
D = 128
A = 16

NC = 2
NS = 16
NW = NC * NS
CHUNK = 128
NCH = -(-E // (NW * CHUNK))
IGRP = 16
NCH = NCH + (-NCH) % IGRP
NGRP = NCH // IGRP
NBUF = 2
E_PAD = NW * NCH * CHUNK
ROWS_PER_TILE = 632
N_ACC = ROWS_PER_TILE * NS


def _sc_segment_sums(x0, x1, x2, srcs, dsts, zeros):
    mesh = plsc.VectorSubcoreMesh(core_axis_name="c", subcore_axis_name="s")

    @functools.partial(
        pl.kernel,
        mesh=mesh,
        out_type=jax.ShapeDtypeStruct((NC, R, N_ACC, D), jnp.float32),
        scratch_types=[
            pltpu.VMEM((IGRP, CHUNK), jnp.int32),
            pltpu.VMEM((IGRP, CHUNK), jnp.int32),
            pltpu.VMEM((NBUF, CHUNK, D), jnp.float32),
            pltpu.VMEM_SHARED((N_ACC, D), jnp.float32),
        ] + [pltpu.SemaphoreType.DMA] * NBUF,
    )
    def seg(x0_hbm, x1_hbm, x2_hbm, srcs_hbm, dsts_hbm, zeros_hbm, out_hbm,
            src_v, dst_v, rows_v, acc, *sems):
        c = lax.axis_index("c")
        s = lax.axis_index("s")
        wid = s * NC + c
        row0 = s * ROWS_PER_TILE
        tables = (x0_hbm, x1_hbm, x2_hbm)
        for r in range(R):
            pltpu.sync_copy(zeros_hbm.at[pl.ds(row0, ROWS_PER_TILE)],
                            acc.at[pl.ds(row0, ROWS_PER_TILE)])
            plsc.subcore_barrier()

            def group(g, carry):
                pltpu.sync_copy(srcs_hbm.at[r, wid, pl.ds(g * IGRP, IGRP)],
                                src_v)
                pltpu.sync_copy(dsts_hbm.at[r, wid, pl.ds(g * IGRP, IGRP)],
                                dst_v)
                for b in range(NBUF):
                    pltpu.async_copy(tables[r].at[src_v.at[b]], rows_v.at[b],
                                     sems[b])

                def stage(i, carry2):
                    j0 = i * NBUF
                    for b in range(NBUF):
                        j = j0 + b
                        pltpu.make_async_copy(tables[r].at[src_v.at[j]],
                                              rows_v.at[b], sems[b]).wait()
                        pltpu.sync_copy(rows_v.at[b], acc.at[dst_v.at[j]],
                                        add=True)
                        nxt = j + NBUF

                        @pl.when(nxt < IGRP)
                        def _():
                            pltpu.async_copy(tables[r].at[src_v.at[nxt]],
                                             rows_v.at[b], sems[b])
                    return carry2

                lax.fori_loop(0, IGRP // NBUF, stage, 0)
                return carry

            lax.fori_loop(0, NGRP, group, 0)
            plsc.subcore_barrier()
            pltpu.sync_copy(acc.at[pl.ds(row0, ROWS_PER_TILE)],
                            out_hbm.at[c, r, pl.ds(row0, ROWS_PER_TILE)])

    return seg(x0, x1, x2, srcs, dsts, zeros)


def _bdot(a, b):
    return jnp.dot(a.astype(jnp.bfloat16), b.astype(jnp.bfloat16),
                   preferred_element_type=jnp.float32)

def _tc_layer_body(x_ref, a0_ref, a1_ref, w1_ref, b1_ref, w2_ref, b2_ref,
                   ws1_ref, ws2_ref, out_ref, *, last):
    hs = []
    logits = []
    for r in range(R):
        xr = x_ref[0] if x_ref.shape[0] == 1 else x_ref[r]
        t = xr + a0_ref[r] + a1_ref[r]
        h = jnp.maximum(_bdot(t, w1_ref[:]) + b1_ref[:], 0.0)
        h = jnp.maximum(_bdot(h, w2_ref[:]) + b2_ref[:], 0.0)
        s = jnp.tanh(_bdot(h, ws1_ref[r]))
        logits.append(_bdot(s, ws2_ref[r]))
        hs.append(h)
    m = jnp.maximum(jnp.maximum(logits[0], logits[1]), logits[2])
    e = [jnp.exp(l - m) for l in logits]
    den = e[0] + e[1] + e[2]
    for rp in range(R):
        a = e[rp] / den
        o = a[:, 0:1] * hs[0] + a[:, 1:2] * hs[1] + a[:, 2:3] * hs[2]
        if last:
            out_ref[:, rp, :] = o
        else:
            out_ref[rp] = o


def _tc_layer(x, agg0, agg1, W1, b1, W2, b2, Ws1, Ws2, *, last):
    B = 512
    grid = (-(-N // B),)
    rx = x.shape[0]
    in_specs = [
        pl.BlockSpec((rx, B, D), lambda i: (0, i, 0)),
        pl.BlockSpec((R, B, D), lambda i: (0, i, 0)),
        pl.BlockSpec((R, B, D), lambda i: (0, i, 0)),
        pl.BlockSpec((D, D), lambda i: (0, 0)),
        pl.BlockSpec((1, D), lambda i: (0, 0)),
        pl.BlockSpec((D, D), lambda i: (0, 0)),
        pl.BlockSpec((1, D), lambda i: (0, 0)),
        pl.BlockSpec((R, D, A), lambda i: (0, 0, 0)),
        pl.BlockSpec((R, A, R), lambda i: (0, 0, 0)),
    ]
    if last:
        out_spec = pl.BlockSpec((B, R, D), lambda i: (i, 0, 0))
        out_shape = jax.ShapeDtypeStruct((N, R, D), jnp.float32)
    else:
        out_spec = pl.BlockSpec((R, B, D), lambda i: (0, i, 0))
        out_shape = jax.ShapeDtypeStruct((R, N, D), jnp.float32)
    return pl.pallas_call(
        functools.partial(_tc_layer_body, last=last),
        grid=grid,
        in_specs=in_specs,
        out_specs=out_spec,
        out_shape=out_shape,
        compiler_params=pltpu.CompilerParams(
            dimension_semantics=("arbitrary",)),
    )(x, agg0, agg1, W1, b1.reshape(1, D), W2, b2.reshape(1, D), Ws1, Ws2)


def kernel(feat, edge_index, W1_0, b1_0, W2_0, b2_0, Ws1_0, Ws2_0,
           W1_1, b1_1, W2_1, b2_1, Ws1_1, Ws2_1):
    src = edge_index[:, 0, :]
    dst = edge_index[:, 1, :]
    pad = E_PAD - E
    src = jnp.pad(src, ((0, 0), (0, pad)), constant_values=0)
    dst = jnp.pad(dst, ((0, 0), (0, pad)), constant_values=N)
    srcs = src.reshape(R, NW, NCH, CHUNK)
    dsts = dst.reshape(R, NW, NCH, CHUNK)
    zeros = jnp.zeros((N_ACC, D), jnp.float32)

    agg = _sc_segment_sums(feat, feat, feat, srcs, dsts, zeros)
    agg = agg[:, :, :N, :]
    h = _tc_layer(feat[None], agg[0], agg[1], W1_0, b1_0, W2_0, b2_0,
                  Ws1_0, Ws2_0, last=False)
    agg = _sc_segment_sums(h[0], h[1], h[2], srcs, dsts, zeros)
    agg = agg[:, :, :N, :]
    out = _tc_layer(h, agg[0], agg[1], W1_1, b1_1, W2_1, b2_1,
                    Ws1_1, Ws2_1, last=True)
    return out
